# baseline (speedup 1.0000x reference)
import functools

import jax
import jax.numpy as jnp
from jax import lax
from jax.experimental import pallas as pl
from jax.experimental.pallas import tpu as pltpu
from jax.experimental.pallas import tpu_sc as plsc

_LANES = 16
_CHUNK = 2048


def _sc_vegas_body(dim, ninc, ppw, chunk, nchunks, ncores,
                   u_hbm, grid_hbm, inc_hbm, xl_hbm, jo_hbm,
                   x_hbm, jac_hbm,
                   grid_v, inc_v, xl_v, jo_v, u_v, x_v, jac_v):
    wid = lax.axis_index("s") * ncores + lax.axis_index("c")
    base = wid * ppw
    pltpu.sync_copy(grid_hbm, grid_v)
    pltpu.sync_copy(inc_hbm, inc_v)
    pltpu.sync_copy(xl_hbm, xl_v)
    pltpu.sync_copy(jo_hbm, jo_v)
    iot = lax.iota(jnp.int32, _LANES)
    fninc = float(ninc)

    def do_group(i, _):
        rows = i * _LANES + iot
        jac = jnp.full((_LANES,), 1.0, jnp.float32)
        for d in range(dim):
            dd = jnp.full((_LANES,), d, jnp.int32)
            ud = plsc.load_gather(u_v, [rows, dd])
            un = ud * fninc
            iu = un.astype(jnp.int32)
            du = un - iu.astype(jnp.float32)
            msk = iu < ninc
            iuc = jnp.minimum(jnp.maximum(iu, 0), ninc - 1)
            g = plsc.load_gather(grid_v, [iuc + (d * (ninc + 1))])
            ig = plsc.load_gather(inc_v, [iuc + (d * ninc)])
            xd = jnp.where(msk, g + ig * du, xl_v[d])
            jac = jac * jnp.where(msk, ig * fninc, jo_v[d])
            plsc.store_scatter(x_v, [rows, dd], xd)
        jac_v[pl.ds(i * _LANES, _LANES)] = jac
        return 0

    def do_chunk(k, _):
        row0 = base + k * chunk
        pltpu.sync_copy(u_hbm.at[pl.ds(row0, chunk), :], u_v)
        lax.fori_loop(0, chunk // _LANES, do_group, 0)
        pltpu.sync_copy(x_v, x_hbm.at[pl.ds(row0, chunk), :])
        pltpu.sync_copy(jac_v, jac_hbm.at[pl.ds(row0, chunk)])
        return 0

    lax.fori_loop(0, nchunks, do_chunk, 0)


def _log_body(j_ref, o_ref):
    o_ref[...] = jnp.log(j_ref[...])


def _tc_log(jacp):
    n = jacp.shape[0]
    rows = n // 128
    brows = min(2048, rows)
    j2 = jacp.reshape(rows, 128)
    out = pl.pallas_call(
        _log_body,
        grid=(rows // brows,),
        in_specs=[pl.BlockSpec((brows, 128), lambda i: (i, 0))],
        out_specs=pl.BlockSpec((brows, 128), lambda i: (i, 0)),
        out_shape=jax.ShapeDtypeStruct((rows, 128), jnp.float32),
    )(j2)
    return out.reshape(n)


def kernel(u, grid, inc):
    n, dim = u.shape
    ninc = grid.shape[1] - 1
    info = plsc.get_sparse_core_info()
    ncores, nsub = info.num_cores, info.num_subcores
    nw = ncores * nsub
    ppw = n // nw
    chunk = min(_CHUNK, ppw)
    nchunks = ppw // chunk

    xlast = jnp.broadcast_to(grid[:, -1:], (dim, _LANES))
    jout = jnp.broadcast_to(inc[:, -1:] * float(ninc), (dim, _LANES))
    grid_f = grid.reshape(dim * (ninc + 1))
    inc_f = inc.reshape(dim * ninc)

    mesh = plsc.VectorSubcoreMesh(core_axis_name="c", subcore_axis_name="s")
    run = functools.partial(
        pl.kernel,
        out_type=[
            jax.ShapeDtypeStruct((n, dim), jnp.float32),
            jax.ShapeDtypeStruct((n,), jnp.float32),
        ],
        mesh=mesh,
        compiler_params=pltpu.CompilerParams(use_tc_tiling_on_sc=False,
                                             needs_layout_passes=False),
        scratch_types=[
            pltpu.VMEM((dim * (ninc + 1),), jnp.float32),
            pltpu.VMEM((dim * ninc,), jnp.float32),
            pltpu.VMEM((dim, _LANES), jnp.float32),
            pltpu.VMEM((dim, _LANES), jnp.float32),
            pltpu.VMEM((chunk, dim), jnp.float32),
            pltpu.VMEM((chunk, dim), jnp.float32),
            pltpu.VMEM((chunk,), jnp.float32),
        ],
    )(functools.partial(_sc_vegas_body, dim, ninc, ppw, chunk, nchunks,
                        ncores))

    x, jacp = run(u, grid_f, inc_f, xlast, jout)
    return x, _tc_log(jacp)


# speedup vs baseline: 1.3671x; 1.3671x over previous
import functools

import jax
import jax.numpy as jnp
from jax import lax
from jax.experimental import pallas as pl
from jax.experimental.pallas import tpu as pltpu
from jax.experimental.pallas import tpu_sc as plsc

_LANES = 16
_CHUNK = 2048


def _sc_vegas_body(dim, ninc, ppw, chunk, nchunks, ncores,
                   u_hbm, grid_hbm, inc_hbm, xl_hbm, jo_hbm,
                   x_hbm, jac_hbm,
                   grid_v, inc_v, xl_v, jo_v, u_v, x_v, jac_v):
    wid = lax.axis_index("s") * ncores + lax.axis_index("c")
    base = wid * ppw
    pltpu.sync_copy(grid_hbm, grid_v)
    pltpu.sync_copy(inc_hbm, inc_v)
    pltpu.sync_copy(xl_hbm, xl_v)
    pltpu.sync_copy(jo_hbm, jo_v)
    fninc = float(ninc)

    def do_group(i, _):
        o = i * _LANES
        jac = jnp.full((_LANES,), 1.0, jnp.float32)
        for d in range(dim):
            ud = u_v[d, pl.ds(o, _LANES)]
            un = ud * fninc
            iu = un.astype(jnp.int32)
            du = un - iu.astype(jnp.float32)
            msk = iu < ninc
            iuc = jnp.minimum(jnp.maximum(iu, 0), ninc - 1)
            g = plsc.load_gather(grid_v, [iuc + (d * (ninc + 1))])
            ig = plsc.load_gather(inc_v, [iuc + (d * ninc)])
            xd = jnp.where(msk, g + ig * du, xl_v[d])
            jac = jac * jnp.where(msk, ig * fninc, jo_v[d])
            x_v[d, pl.ds(o, _LANES)] = xd
        jac_v[pl.ds(o, _LANES)] = jac
        return 0

    def do_chunk(k, _):
        col0 = base + k * chunk
        pltpu.sync_copy(u_hbm.at[:, pl.ds(col0, chunk)], u_v)
        lax.fori_loop(0, chunk // _LANES, do_group, 0)
        pltpu.sync_copy(x_v, x_hbm.at[:, pl.ds(col0, chunk)])
        pltpu.sync_copy(jac_v, jac_hbm.at[pl.ds(col0, chunk)])
        return 0

    lax.fori_loop(0, nchunks, do_chunk, 0)


def _log_body(j_ref, o_ref):
    o_ref[...] = jnp.log(j_ref[...])


def _tc_log(jacp):
    n = jacp.shape[0]
    rows = n // 128
    brows = min(2048, rows)
    j2 = jacp.reshape(rows, 128)
    out = pl.pallas_call(
        _log_body,
        grid=(rows // brows,),
        in_specs=[pl.BlockSpec((brows, 128), lambda i: (i, 0))],
        out_specs=pl.BlockSpec((brows, 128), lambda i: (i, 0)),
        out_shape=jax.ShapeDtypeStruct((rows, 128), jnp.float32),
    )(j2)
    return out.reshape(n)


def kernel(u, grid, inc):
    n, dim = u.shape
    ninc = grid.shape[1] - 1
    info = plsc.get_sparse_core_info()
    ncores, nsub = info.num_cores, info.num_subcores
    nw = ncores * nsub
    ppw = n // nw
    chunk = min(_CHUNK, ppw)
    nchunks = ppw // chunk

    xlast = jnp.broadcast_to(grid[:, -1:], (dim, _LANES))
    jout = jnp.broadcast_to(inc[:, -1:] * float(ninc), (dim, _LANES))
    grid_f = grid.reshape(dim * (ninc + 1))
    inc_f = inc.reshape(dim * ninc)

    mesh = plsc.VectorSubcoreMesh(core_axis_name="c", subcore_axis_name="s")
    run = functools.partial(
        pl.kernel,
        out_type=[
            jax.ShapeDtypeStruct((dim, n), jnp.float32),
            jax.ShapeDtypeStruct((n,), jnp.float32),
        ],
        mesh=mesh,
        compiler_params=pltpu.CompilerParams(use_tc_tiling_on_sc=False,
                                             needs_layout_passes=False),
        scratch_types=[
            pltpu.VMEM((dim * (ninc + 1),), jnp.float32),
            pltpu.VMEM((dim * ninc,), jnp.float32),
            pltpu.VMEM((dim, _LANES), jnp.float32),
            pltpu.VMEM((dim, _LANES), jnp.float32),
            pltpu.VMEM((dim, chunk), jnp.float32),
            pltpu.VMEM((dim, chunk), jnp.float32),
            pltpu.VMEM((chunk,), jnp.float32),
        ],
    )(functools.partial(_sc_vegas_body, dim, ninc, ppw, chunk, nchunks,
                        ncores))

    x_t, jacp = run(u.T, grid_f, inc_f, xlast, jout)
    return x_t.T, _tc_log(jacp)


# speedup vs baseline: 8.1972x; 5.9959x over previous
import functools

import jax
import jax.numpy as jnp
from jax import lax
from jax.experimental import pallas as pl
from jax.experimental.pallas import tpu as pltpu
from jax.experimental.pallas import tpu_sc as plsc

_LANES = 16
_CHUNK = 2048


def _sc_vegas_body(dim, ninc, ppw, chunk, nchunks, ncores,
                   u_hbm, grid_hbm, inc_hbm,
                   x_hbm, jac_hbm,
                   grid_v, inc_v, u_v, x_v, jac_v):
    wid = lax.axis_index("s") * ncores + lax.axis_index("c")
    base = wid * ppw
    pltpu.sync_copy(grid_hbm, grid_v)
    pltpu.sync_copy(inc_hbm, inc_v)
    fninc = float(ninc)

    def do_group(o):
        jfs = []
        for d in range(dim):
            ud = u_v[d, pl.ds(o, _LANES)]
            un = ud * fninc
            iu = un.astype(jnp.int32)
            du = un - iu.astype(jnp.float32)
            idx = jnp.minimum(iu, ninc) + (d * (ninc + 1))
            g = plsc.load_gather(grid_v, [idx])
            ig = plsc.load_gather(inc_v, [idx])
            x_v[d, pl.ds(o, _LANES)] = g + ig * du
            jfs.append(ig * fninc)
        while len(jfs) > 1:
            jfs = [a * b for a, b in zip(jfs[::2], jfs[1::2])]
        jac_v[pl.ds(o, _LANES)] = jfs[0]

    def do_chunk(k, _):
        col0 = base + k * chunk
        pltpu.sync_copy(u_hbm.at[:, pl.ds(col0, chunk)], u_v)
        plsc.parallel_loop(0, chunk, step=_LANES, unroll=4)(do_group)
        def do_tile(t, _):
            pltpu.sync_copy(x_v.at[:, pl.ds(t * 128, 128)],
                            x_hbm.at[col0 // 128 + t])
            return 0
        lax.fori_loop(0, chunk // 128, do_tile, 0)
        pltpu.sync_copy(jac_v, jac_hbm.at[pl.ds(col0, chunk)])
        return 0

    lax.fori_loop(0, nchunks, do_chunk, 0)


def _log_body(j_ref, o_ref):
    o_ref[...] = jnp.log(j_ref[...])


def _tc_log(jacp):
    n = jacp.shape[0]
    rows = n // 128
    brows = min(2048, rows)
    j2 = jacp.reshape(rows, 128)
    out = pl.pallas_call(
        _log_body,
        grid=(rows // brows,),
        in_specs=[pl.BlockSpec((brows, 128), lambda i: (i, 0))],
        out_specs=pl.BlockSpec((brows, 128), lambda i: (i, 0)),
        out_shape=jax.ShapeDtypeStruct((rows, 128), jnp.float32),
    )(j2)
    return out.reshape(n)


def kernel(u, grid, inc):
    n, dim = u.shape
    ninc = grid.shape[1] - 1
    info = plsc.get_sparse_core_info()
    ncores, nsub = info.num_cores, info.num_subcores
    nw = ncores * nsub
    ppw = n // nw
    chunk = min(_CHUNK, ppw)
    nchunks = ppw // chunk

    grid_f = grid.reshape(dim * (ninc + 1))
    inc_f = jnp.concatenate([inc, inc[:, -1:]], axis=1).reshape(
        dim * (ninc + 1))

    mesh = plsc.VectorSubcoreMesh(core_axis_name="c", subcore_axis_name="s")
    run = functools.partial(
        pl.kernel,
        out_type=[
            jax.ShapeDtypeStruct((n // 128, dim, 128), jnp.float32),
            jax.ShapeDtypeStruct((n,), jnp.float32),
        ],
        mesh=mesh,
        compiler_params=pltpu.CompilerParams(use_tc_tiling_on_sc=False,
                                             needs_layout_passes=False),
        scratch_types=[
            pltpu.VMEM((dim * (ninc + 1),), jnp.float32),
            pltpu.VMEM((dim * (ninc + 1),), jnp.float32),
            pltpu.VMEM((dim, chunk), jnp.float32),
            pltpu.VMEM((dim, chunk), jnp.float32),
            pltpu.VMEM((chunk,), jnp.float32),
        ],
    )(functools.partial(_sc_vegas_body, dim, ninc, ppw, chunk, nchunks,
                        ncores))

    x_f, jacp = run(u.T, grid_f, inc_f)
    x = x_f.transpose(0, 2, 1).reshape(n, dim)
    return x, _tc_log(jacp)


# speedup vs baseline: 10.8848x; 1.3279x over previous
import functools

import jax
import jax.numpy as jnp
from jax import lax
from jax.experimental import pallas as pl
from jax.experimental.pallas import tpu as pltpu
from jax.experimental.pallas import tpu_sc as plsc

_LANES = 16
_CHUNK = 2048


def _sc_vegas_body(dim, ninc, ppw, chunk, nchunks, ncores,
                   u_hbm, grid_hbm, inc_hbm,
                   x_hbm, jac_hbm,
                   grid_v, inc_v, u_v, x_v, jac_v, usem, xsem, jsem):
    wid = lax.axis_index("s") * ncores + lax.axis_index("c")
    base = wid * ppw
    pltpu.sync_copy(grid_hbm, grid_v)
    pltpu.sync_copy(inc_hbm, inc_v)
    fninc = float(ninc)
    ntiles = chunk // 128

    def u_copy(k, b):
        return pltpu.make_async_copy(
            u_hbm.at[:, pl.ds(base + k * chunk, chunk)], u_v.at[b],
            usem.at[b])

    def start_out(k, b):
        col0 = base + k * chunk
        def do_tile(t, _):
            pltpu.async_copy(x_v.at[b, :, pl.ds(t * 128, 128)],
                             x_hbm.at[col0 // 128 + t], xsem.at[b])
            return 0
        lax.fori_loop(0, ntiles, do_tile, 0)
        pltpu.async_copy(jac_v.at[b], jac_hbm.at[pl.ds(col0, chunk)],
                         jsem.at[b])

    def wait_out(b):
        def do_tile(t, _):
            pltpu.make_async_copy(x_v.at[b, :, pl.ds(0, 128)],
                                  x_hbm.at[0], xsem.at[b]).wait()
            return 0
        lax.fori_loop(0, ntiles, do_tile, 0)
        pltpu.make_async_copy(jac_v.at[b], jac_hbm.at[pl.ds(0, chunk)],
                              jsem.at[b]).wait()

    def do_group_in(b):
        def do_group(o):
            jfs = []
            for d in range(dim):
                ud = u_v[b, d, pl.ds(o, _LANES)]
                un = ud * fninc
                iu = un.astype(jnp.int32)
                du = un - iu.astype(jnp.float32)
                idx = jnp.minimum(iu, ninc) + (d * (ninc + 1))
                g = plsc.load_gather(grid_v, [idx])
                ig = plsc.load_gather(inc_v, [idx])
                x_v[b, d, pl.ds(o, _LANES)] = g + ig * du
                jfs.append(ig * fninc)
            while len(jfs) > 1:
                jfs = [a * b2 for a, b2 in zip(jfs[::2], jfs[1::2])]
            jac_v[b, pl.ds(o, _LANES)] = jfs[0]
        return do_group

    u_copy(0, 0).start()

    def do_pair(kk, _):
        for b in range(2):
            k = kk * 2 + b
            u_copy(k, b).wait()

            @pl.when(k + 1 < nchunks)
            def _():
                u_copy(k + 1, 1 - b).start()

            @pl.when(k >= 2)
            def _():
                wait_out(b)

            plsc.parallel_loop(0, chunk, step=_LANES,
                               unroll=4)(do_group_in(b))
            start_out(k, b)
        return 0

    lax.fori_loop(0, nchunks // 2, do_pair, 0)
    wait_out(0)
    wait_out(1)


def _log_body(j_ref, o_ref):
    o_ref[...] = jnp.log(j_ref[...])


def _tc_log(jacp):
    n = jacp.shape[0]
    rows = n // 128
    brows = min(2048, rows)
    j2 = jacp.reshape(rows, 128)
    out = pl.pallas_call(
        _log_body,
        grid=(rows // brows,),
        in_specs=[pl.BlockSpec((brows, 128), lambda i: (i, 0))],
        out_specs=pl.BlockSpec((brows, 128), lambda i: (i, 0)),
        out_shape=jax.ShapeDtypeStruct((rows, 128), jnp.float32),
    )(j2)
    return out.reshape(n)


def kernel(u, grid, inc):
    n, dim = u.shape
    ninc = grid.shape[1] - 1
    info = plsc.get_sparse_core_info()
    ncores, nsub = info.num_cores, info.num_subcores
    nw = ncores * nsub
    ppw = n // nw
    chunk = min(_CHUNK, ppw)
    nchunks = ppw // chunk

    grid_f = grid.reshape(dim * (ninc + 1))
    inc_f = jnp.concatenate([inc, inc[:, -1:]], axis=1).reshape(
        dim * (ninc + 1))

    mesh = plsc.VectorSubcoreMesh(core_axis_name="c", subcore_axis_name="s")
    run = functools.partial(
        pl.kernel,
        out_type=[
            jax.ShapeDtypeStruct((n // 128, dim, 128), jnp.float32),
            jax.ShapeDtypeStruct((n,), jnp.float32),
        ],
        mesh=mesh,
        compiler_params=pltpu.CompilerParams(use_tc_tiling_on_sc=False,
                                             needs_layout_passes=False),
        scratch_types=[
            pltpu.VMEM((dim * (ninc + 1),), jnp.float32),
            pltpu.VMEM((dim * (ninc + 1),), jnp.float32),
            pltpu.VMEM((2, dim, chunk), jnp.float32),
            pltpu.VMEM((2, dim, chunk), jnp.float32),
            pltpu.VMEM((2, chunk), jnp.float32),
            pltpu.SemaphoreType.DMA((2,)),
            pltpu.SemaphoreType.DMA((2,)),
            pltpu.SemaphoreType.DMA((2,)),
        ],
    )(functools.partial(_sc_vegas_body, dim, ninc, ppw, chunk, nchunks,
                        ncores))

    x_f, jacp = run(u.T, grid_f, inc_f)
    x = x_f.transpose(0, 2, 1).reshape(n, dim)
    return x, _tc_log(jacp)


# speedup vs baseline: 13.7723x; 1.2653x over previous
import functools

import jax
import jax.numpy as jnp
from jax import lax
from jax.experimental import pallas as pl
from jax.experimental.pallas import tpu as pltpu
from jax.experimental.pallas import tpu_sc as plsc

_LANES = 16
_CHUNK = 2048


def _sc_vegas_body(dim, ninc, ppw, chunk, nchunks, ncores,
                   u_hbm, grid_hbm, inc_hbm,
                   x_hbm, jac_hbm,
                   grid_v, inc_v, u_v, x_v, jac_v, usem, xsem, jsem):
    wid = lax.axis_index("s") * ncores + lax.axis_index("c")
    base = wid * ppw
    pltpu.sync_copy(grid_hbm, grid_v)
    pltpu.sync_copy(inc_hbm, inc_v)
    fninc = float(ninc)
    ntiles = chunk // 128

    def u_copy(k, b):
        t0 = (base + k * chunk) // 128
        return pltpu.make_async_copy(
            u_hbm.at[pl.ds(t0, chunk // 128)], u_v.at[b], usem.at[b])

    def start_out(k, b):
        col0 = base + k * chunk
        def do_tile(t, _):
            pltpu.async_copy(x_v.at[b, :, pl.ds(t * 128, 128)],
                             x_hbm.at[col0 // 128 + t], xsem.at[b])
            return 0
        lax.fori_loop(0, ntiles, do_tile, 0)
        pltpu.async_copy(jac_v.at[b], jac_hbm.at[pl.ds(col0, chunk)],
                         jsem.at[b])

    def wait_out(b):
        def do_tile(t, _):
            pltpu.make_async_copy(x_v.at[b, :, pl.ds(0, 128)],
                                  x_hbm.at[0], xsem.at[b]).wait()
            return 0
        lax.fori_loop(0, ntiles, do_tile, 0)
        pltpu.make_async_copy(jac_v.at[b], jac_hbm.at[pl.ds(0, chunk)],
                              jsem.at[b]).wait()

    def do_group_in(b):
        def do_group(o):
            t = o >> 7
            c = o & 127
            jfs = []
            for d in range(dim):
                ud = u_v[b, t, d, pl.ds(c, _LANES)]
                un = ud * fninc
                iu = un.astype(jnp.int32)
                du = un - iu.astype(jnp.float32)
                idx = jnp.minimum(iu, ninc) + (d * (ninc + 1))
                g = plsc.load_gather(grid_v, [idx])
                ig = plsc.load_gather(inc_v, [idx])
                x_v[b, d, pl.ds(o, _LANES)] = g + ig * du
                jfs.append(ig * fninc)
            while len(jfs) > 1:
                jfs = [a * b2 for a, b2 in zip(jfs[::2], jfs[1::2])]
            jac_v[b, pl.ds(o, _LANES)] = jfs[0]
        return do_group

    u_copy(0, 0).start()

    def do_pair(kk, _):
        for b in range(2):
            k = kk * 2 + b
            u_copy(k, b).wait()

            @pl.when(k + 1 < nchunks)
            def _():
                u_copy(k + 1, 1 - b).start()

            @pl.when(k >= 2)
            def _():
                wait_out(b)

            plsc.parallel_loop(0, chunk, step=_LANES,
                               unroll=4)(do_group_in(b))
            start_out(k, b)
        return 0

    lax.fori_loop(0, nchunks // 2, do_pair, 0)
    wait_out(0)
    wait_out(1)


def _log_body(j_ref, o_ref):
    o_ref[...] = jnp.log(j_ref[...])


def _tc_log(jacp):
    n = jacp.shape[0]
    rows = n // 128
    brows = min(2048, rows)
    j2 = jacp.reshape(rows, 128)
    out = pl.pallas_call(
        _log_body,
        grid=(rows // brows,),
        in_specs=[pl.BlockSpec((brows, 128), lambda i: (i, 0))],
        out_specs=pl.BlockSpec((brows, 128), lambda i: (i, 0)),
        out_shape=jax.ShapeDtypeStruct((rows, 128), jnp.float32),
    )(j2)
    return out.reshape(n)


def kernel(u, grid, inc):
    n, dim = u.shape
    ninc = grid.shape[1] - 1
    info = plsc.get_sparse_core_info()
    ncores, nsub = info.num_cores, info.num_subcores
    nw = ncores * nsub
    ppw = n // nw
    chunk = min(_CHUNK, ppw)
    nchunks = ppw // chunk

    grid_f = grid.reshape(dim * (ninc + 1))
    inc_f = jnp.concatenate([inc, inc[:, -1:]], axis=1).reshape(
        dim * (ninc + 1))

    mesh = plsc.VectorSubcoreMesh(core_axis_name="c", subcore_axis_name="s")
    run = functools.partial(
        pl.kernel,
        out_type=[
            jax.ShapeDtypeStruct((n // 128, dim, 128), jnp.float32),
            jax.ShapeDtypeStruct((n,), jnp.float32),
        ],
        mesh=mesh,
        compiler_params=pltpu.CompilerParams(use_tc_tiling_on_sc=False,
                                             needs_layout_passes=False),
        scratch_types=[
            pltpu.VMEM((dim * (ninc + 1),), jnp.float32),
            pltpu.VMEM((dim * (ninc + 1),), jnp.float32),
            pltpu.VMEM((2, chunk // 128, dim, 128), jnp.float32),
            pltpu.VMEM((2, dim, chunk), jnp.float32),
            pltpu.VMEM((2, chunk), jnp.float32),
            pltpu.SemaphoreType.DMA((2,)),
            pltpu.SemaphoreType.DMA((2,)),
            pltpu.SemaphoreType.DMA((2,)),
        ],
    )(functools.partial(_sc_vegas_body, dim, ninc, ppw, chunk, nchunks,
                        ncores))

    u3 = u.reshape(n // 128, 128, dim).transpose(0, 2, 1)
    x_f, jacp = run(u3, grid_f, inc_f)
    x = x_f.transpose(0, 2, 1).reshape(n, dim)
    return x, _tc_log(jacp)
